# baseline (speedup 1.0000x reference)
import functools

import jax
import jax.numpy as jnp
from jax import lax
from jax.experimental import pallas as pl
from jax.experimental.pallas import tpu as pltpu
from jax.experimental.pallas import tpu_sc as plsc

B = 4096
D = 128
E = 320000
R = 2
NC = 2
NS = 16
L = 16
CE = E // NS
KB = 128
CEB = CE + 2 * KB
TR = 4112
STRIPE = B // NS
NW = NC * NS

_mesh = plsc.VectorSubcoreMesh(
    core_axis_name="c", subcore_axis_name="s", num_cores=NC, num_subcores=NS)


@functools.partial(
    pl.kernel,
    out_type=(
        jax.ShapeDtypeStruct((NW, 1, 2 * CEB), jnp.int32),
        jax.ShapeDtypeStruct((NW, 1, 16), jnp.int32),
        jax.ShapeDtypeStruct((NW, 1, B), jnp.int32),
    ),
    mesh=_mesh,
    compiler_params=pltpu.CompilerParams(needs_layout_passes=False),
    scratch_types=(
        pltpu.VMEM((CE,), jnp.int32),
        pltpu.VMEM((CE,), jnp.int32),
        pltpu.VMEM((2 * CEB,), jnp.int32),
        pltpu.VMEM((B,), jnp.int32),
        pltpu.VMEM((B,), jnp.int32),
        pltpu.VMEM((B,), jnp.int32),
        pltpu.VMEM((B // NS,), jnp.int32),
        pltpu.VMEM((B // NS,), jnp.int32),
        pltpu.VMEM((B // NS,), jnp.int32),
        pltpu.VMEM((16,), jnp.int32),
        pltpu.VMEM_SHARED((NS + 1, 1, B), jnp.int32),
        pltpu.SemaphoreType.DMA,
        pltpu.SemaphoreType.DMA,
    ),
)
def _preprocess(edges, idx_out, cnt_out, deg_out,
                src_v, dst_v, idx_v, pres_v, deg_v, rank_v, tmp_v,
                tmp2_v, acc_v, misc_v, pres_sh, csem0, csem1):
    c = lax.axis_index("c")
    s = lax.axis_index("s")
    w = c * NS + s
    iota16 = lax.iota(jnp.int32, 16)
    one16 = jnp.ones((L,), jnp.int32)
    zero16 = jnp.zeros((L,), jnp.int32)

    pltpu.sync_copy(edges.at[2 * c, s, 0], src_v)
    pltpu.sync_copy(edges.at[2 * c + 1, s, 0], dst_v)

    def zero_body(i, _):
        pres_v[pl.ds(i * 16, 16)] = zero16
        deg_v[pl.ds(i * 16, 16)] = zero16
        return 0
    lax.fori_loop(0, B // 16, zero_body, 0)

    def pres_body(i, _):
        for u in range(2):
            sv = src_v[pl.ds(i * 32 + u * 16, 16)]
            dv = dst_v[pl.ds(i * 32 + u * 16, 16)]
            m = (sv < B) & (dv < B)
            svc = jnp.where(m, sv, 0)
            dvc = jnp.where(m, dv, 0)
            plsc.store_scatter(pres_v, [svc], one16, mask=m)
            plsc.store_scatter(pres_v, [dvc], one16, mask=m)
        return 0
    lax.fori_loop(0, CE // 32, pres_body, 0)

    SEG = B // NS
    pltpu.sync_copy(pres_v, pres_sh.at[s, 0])
    plsc.subcore_barrier()

    def z16(i, _):
        acc_v[pl.ds(i * 16, 16)] = zero16
        return 0
    lax.fori_loop(0, SEG // 16, z16, 0)
    tmps = (tmp_v, tmp2_v)
    csems = (csem0, csem1)
    pltpu.async_copy(pres_sh.at[0, 0, pl.ds(s * SEG, SEG)], tmp_v, csem0)
    for t in range(NS):
        pltpu.make_async_copy(pres_sh.at[t, 0, pl.ds(s * SEG, SEG)],
                              tmps[t % 2], csems[t % 2]).wait()
        if t + 1 < NS:
            pltpu.async_copy(pres_sh.at[t + 1, 0, pl.ds(s * SEG, SEG)],
                             tmps[(t + 1) % 2], csems[(t + 1) % 2])
        tref = tmps[t % 2]

        def add16(k, _, tref=tref):
            acc_v[pl.ds(k * 16, 16)] = (acc_v[pl.ds(k * 16, 16)]
                                        + tref[pl.ds(k * 16, 16)])
            return 0
        lax.fori_loop(0, SEG // 16, add16, 0)
    pltpu.sync_copy(acc_v, pres_sh.at[NS, 0, pl.ds(s * SEG, SEG)])
    plsc.subcore_barrier()
    pltpu.sync_copy(pres_sh.at[NS, 0], pres_v)

    def rank_body(i, carry):
        v = pres_v[pl.ds(i * 16, 16)]
        ind = (v > 0).astype(jnp.int32)
        incl = plsc.cumsum(ind)
        rank_v[pl.ds(i * 16, 16)] = carry + incl - ind
        return carry + jnp.sum(ind)
    lax.fori_loop(0, B // 16, rank_body, jnp.int32(0))

    def edge_body(i, cnt):
        sv = src_v[pl.ds(i * 16, 16)]
        dv = dst_v[pl.ds(i * 16, 16)]
        m = (sv < B) & (dv < B)
        svc = jnp.where(m, sv, 0)
        dvc = jnp.where(m, dv, 0)
        ns = plsc.load_gather(rank_v, [svc], mask=m)
        nd = plsc.load_gather(rank_v, [dvc], mask=m)
        plsc.addupdate_scatter(deg_v, [nd], one16, mask=m)
        mi = m.astype(jnp.int32)
        pos = cnt + plsc.cumsum(mi) - mi
        pos2 = ((pos >> 7) << 8) | (pos & 127)
        plsc.store_scatter(idx_v, [pos2], ns, mask=m)
        plsc.store_scatter(idx_v, [pos2 + KB], nd, mask=m)
        return cnt + jnp.sum(mi)
    cnt = lax.fori_loop(0, CE // 16, edge_body, jnp.int32(0))

    dum = jnp.full((16,), B, jnp.int32)
    for j in range(8):
        idx = cnt + j * 16 + iota16
        idx2 = ((idx >> 7) << 8) | (idx & 127)
        plsc.store_scatter(idx_v, [idx2], dum)
        plsc.store_scatter(idx_v, [idx2 + KB], dum)
    nb = (cnt + KB - 1) // KB
    misc_v[...] = jnp.full((16,), nb, jnp.int32)

    pltpu.sync_copy(idx_v, idx_out.at[w, 0])
    pltpu.sync_copy(misc_v, cnt_out.at[w, 0])
    pltpu.sync_copy(deg_v, deg_out.at[w, 0])


@functools.partial(
    pl.kernel,
    out_type=jax.ShapeDtypeStruct((NC, B, 1, D), jnp.float32),
    mesh=_mesh,
    compiler_params=pltpu.CompilerParams(needs_layout_passes=False),
    scratch_types=(
        pltpu.VMEM((16,), jnp.int32),
        pltpu.VMEM((2 * KB,), jnp.int32),
        pltpu.VMEM((2 * KB,), jnp.int32),
        pltpu.VMEM((2 * KB,), jnp.int32),
        pltpu.VMEM((2 * KB,), jnp.int32),
        pltpu.VMEM((2 * KB,), jnp.int32),
        pltpu.VMEM((KB, 1, D), jnp.float32),
        pltpu.VMEM((KB, 1, D), jnp.float32),
        pltpu.VMEM((KB, 1, D), jnp.float32),
        pltpu.VMEM((KB, 1, D), jnp.float32),
        pltpu.VMEM((KB, 1, D), jnp.float32),
        pltpu.VMEM_SHARED((TR, 1, D), jnp.float32),
        pltpu.SemaphoreType.DMA,
        pltpu.SemaphoreType.DMA,
        pltpu.SemaphoreType.DMA,
        pltpu.SemaphoreType.DMA,
        pltpu.SemaphoreType.DMA,
        pltpu.SemaphoreType.DMA,
        pltpu.SemaphoreType.DMA,
        pltpu.SemaphoreType.DMA,
        pltpu.SemaphoreType.DMA,
        pltpu.SemaphoreType.DMA,
        pltpu.SemaphoreType.DMA,
        pltpu.SemaphoreType.DMA,
        pltpu.SemaphoreType.DMA,
        pltpu.SemaphoreType.DMA,
        pltpu.SemaphoreType.DMA,
    ),
)
def _conv(ytab, idx_in, cnt_in, zeros, out,
          cnt_v, x0, x1, x2, x3, x4, r0, r1, r2, r3, r4, acc_sh,
          i0, i1, i2, i3, i4, g0, g1, g2, g3, g4, s0, s1, s2, s3, s4):
    NBUF = 5
    idxb = (x0, x1, x2, x3, x4)
    rows = (r0, r1, r2, r3, r4)
    isems = (i0, i1, i2, i3, i4)
    gsems = (g0, g1, g2, g3, g4)
    ssems = (s0, s1, s2, s3, s4)
    c = lax.axis_index("c")
    s = lax.axis_index("s")
    w = c * NS + s
    ytab4 = ytab.reshape(NC, TR, 1, D)
    pltpu.sync_copy(zeros.at[pl.ds(s * STRIPE, STRIPE)],
                    acc_sh.at[pl.ds(s * STRIPE, STRIPE)])
    pltpu.sync_copy(cnt_in.at[w, 0], cnt_v)
    nb = jnp.max(cnt_v[pl.ds(0, 16)])
    plsc.subcore_barrier()

    for b in range(NBUF):
        @pl.when(b < nb)
        def _(b=b):
            pltpu.async_copy(idx_in.at[w, 0, pl.ds(b * 2 * KB, 2 * KB)],
                             idxb[b], isems[b])

    def outer(i, _):
        j0 = i * NBUF
        for b in range(NBUF):
            j = j0 + b

            @pl.when(j < nb)
            def _(b=b, j=j):
                pltpu.make_async_copy(idx_in.at[w, 0, pl.ds(0, 2 * KB)],
                                      idxb[b], isems[b]).wait()
                pltpu.async_copy(ytab4.at[c].at[idxb[b].at[pl.ds(0, KB)]],
                                 rows[b], gsems[b]).wait()
                pltpu.async_copy(rows[b],
                                 acc_sh.at[idxb[b].at[pl.ds(KB, KB)]],
                                 ssems[b], add=True).wait()

                @pl.when(j + NBUF < nb)
                def _():
                    pltpu.async_copy(
                        idx_in.at[w, 0, pl.ds((j + NBUF) * 2 * KB, 2 * KB)],
                        idxb[b], isems[b])
        return 0
    lax.fori_loop(0, (nb + NBUF - 1) // NBUF, outer, 0)
    plsc.subcore_barrier()
    pltpu.sync_copy(acc_sh.at[pl.ds(s * STRIPE, STRIPE)],
                    out.at[c, pl.ds(s * STRIPE, STRIPE)])


def _tc_a_body(x_ref, w1_ref, degp_ref, ytab_ref, xw_ref, dinv_ref):
    degs = jnp.sum(degp_ref[...].reshape(R, NS, B), axis=1)
    deg = degs.astype(jnp.float32) + 1.0
    dinv = lax.rsqrt(deg)
    dinv_ref[...] = dinv
    zpad = jnp.zeros((TR - B, D), jnp.float32)
    for r in range(R):
        xw = jnp.dot(x_ref[r], w1_ref[...], preferred_element_type=jnp.float32)
        xw_ref[r] = xw
        ytab_ref[pl.ds(r * TR, B), :] = xw * dinv[r][:, None]
        ytab_ref[pl.ds(r * TR + B, TR - B), :] = zpad


def _tc_b_body(acc_ref, xw1_ref, dinv_ref, b1_ref, g_ref, be_ref, w2_ref,
               ytab_ref, xw2_ref):
    zpad = jnp.zeros((TR - B, D), jnp.float32)
    for r in range(R):
        dinv = dinv_ref[r]
        f1 = (acc_ref[r] * dinv[:, None]
              + xw1_ref[r] * (dinv * dinv)[:, None] + b1_ref[...][None, :])
        mu = jnp.mean(f1, axis=0)
        cen = f1 - mu[None, :]
        var = jnp.mean(cen * cen, axis=0)
        f1n = cen * lax.rsqrt(var + 1e-5)[None, :] * g_ref[...][None, :] \
            + be_ref[...][None, :]
        xw2 = jnp.dot(f1n, w2_ref[...], preferred_element_type=jnp.float32)
        xw2_ref[r] = xw2
        ytab_ref[pl.ds(r * TR, B), :] = xw2 * dinv[:, None]
        ytab_ref[pl.ds(r * TR + B, TR - B), :] = zpad


def _tc_c_body(acc_ref, xw2_ref, dinv_ref, b2_ref, out_ref):
    for r in range(R):
        dinv = dinv_ref[r]
        out_ref[r] = (acc_ref[r] * dinv[:, None]
                      + xw2_ref[r] * (dinv * dinv)[:, None]
                      + b2_ref[...][None, :])


_tc_a = pl.pallas_call(
    _tc_a_body,
    out_shape=(
        jax.ShapeDtypeStruct((R * TR, D), jnp.float32),
        jax.ShapeDtypeStruct((R, B, D), jnp.float32),
        jax.ShapeDtypeStruct((R, B), jnp.float32),
    ),
)

_tc_b = pl.pallas_call(
    _tc_b_body,
    out_shape=(
        jax.ShapeDtypeStruct((R * TR, D), jnp.float32),
        jax.ShapeDtypeStruct((R, B, D), jnp.float32),
    ),
)

_tc_c = pl.pallas_call(
    _tc_c_body,
    out_shape=jax.ShapeDtypeStruct((R, B, D), jnp.float32),
)


def kernel(features_list, multi_r_data, batch_nodes, device,
           W1, b1, gamma, beta, W2, b2):
    del batch_nodes, device
    x2 = features_list[:, :B, :]
    edges = multi_r_data.reshape(2 * R, NS, 1, CE)
    idx, cnt, degp = _preprocess(edges)
    ytab1, xw1, dinv = _tc_a(x2, W1, degp)
    zeros = jnp.zeros((B, 1, D), jnp.float32)
    acc1 = _conv(ytab1.reshape(R * TR, 1, D), idx, cnt, zeros)
    ytab2, xw2 = _tc_b(acc1.reshape(R, B, D), xw1, dinv, b1, gamma, beta, W2)
    acc2 = _conv(ytab2.reshape(R * TR, 1, D), idx, cnt, zeros)
    f2 = _tc_c(acc2.reshape(R, B, D), xw2, dinv, b2)
    return f2.reshape(B, R * D)


# speedup vs baseline: 1.2709x; 1.2709x over previous
import functools

import jax
import jax.numpy as jnp
from jax import lax
from jax.experimental import pallas as pl
from jax.experimental.pallas import tpu as pltpu
from jax.experimental.pallas import tpu_sc as plsc

B = 4096
D = 128
E = 320000
R = 2
NC = 2
NS = 16
L = 16
CE = E // NS
KB = 128
CEB = CE + 2 * KB
TR = 4112
STRIPE = B // NS
NW = NC * NS

_mesh = plsc.VectorSubcoreMesh(
    core_axis_name="c", subcore_axis_name="s", num_cores=NC, num_subcores=NS)


@functools.partial(
    pl.kernel,
    out_type=(
        jax.ShapeDtypeStruct((NW, 1, 2 * CEB), jnp.int32),
        jax.ShapeDtypeStruct((NW, 1, 16), jnp.int32),
        jax.ShapeDtypeStruct((NW, 1, B), jnp.int32),
    ),
    mesh=_mesh,
    compiler_params=pltpu.CompilerParams(needs_layout_passes=False),
    scratch_types=(
        pltpu.VMEM((CE,), jnp.int32),
        pltpu.VMEM((CE,), jnp.int32),
        pltpu.VMEM((2 * CEB,), jnp.int32),
        pltpu.VMEM((B,), jnp.int32),
        pltpu.VMEM((B,), jnp.int32),
        pltpu.VMEM((B,), jnp.int32),
        pltpu.VMEM((B // NS,), jnp.int32),
        pltpu.VMEM((B // NS,), jnp.int32),
        pltpu.VMEM((B // NS,), jnp.int32),
        pltpu.VMEM((16,), jnp.int32),
        pltpu.VMEM_SHARED((NS + 1, 1, B), jnp.int32),
        pltpu.SemaphoreType.DMA,
        pltpu.SemaphoreType.DMA,
    ),
)
def _preprocess(edges, idx_out, cnt_out, deg_out,
                src_v, dst_v, idx_v, pres_v, deg_v, rank_v, tmp_v,
                tmp2_v, acc_v, misc_v, pres_sh, csem0, csem1):
    c = lax.axis_index("c")
    s = lax.axis_index("s")
    w = c * NS + s
    iota16 = lax.iota(jnp.int32, 16)
    one16 = jnp.ones((L,), jnp.int32)
    zero16 = jnp.zeros((L,), jnp.int32)

    pltpu.sync_copy(edges.at[2 * c, s, 0], src_v)
    pltpu.sync_copy(edges.at[2 * c + 1, s, 0], dst_v)

    def zero_body(i, _):
        pres_v[pl.ds(i * 16, 16)] = zero16
        deg_v[pl.ds(i * 16, 16)] = zero16
        return 0
    lax.fori_loop(0, B // 16, zero_body, 0)

    def pres_body(i, _):
        for u in range(2):
            sv = src_v[pl.ds(i * 32 + u * 16, 16)]
            dv = dst_v[pl.ds(i * 32 + u * 16, 16)]
            m = (sv < B) & (dv < B)
            svc = jnp.where(m, sv, 0)
            dvc = jnp.where(m, dv, 0)
            plsc.store_scatter(pres_v, [svc], one16, mask=m)
            plsc.store_scatter(pres_v, [dvc], one16, mask=m)
        return 0
    lax.fori_loop(0, CE // 32, pres_body, 0)

    SEG = B // NS
    pltpu.sync_copy(pres_v, pres_sh.at[s, 0])
    plsc.subcore_barrier()

    def z16(i, _):
        acc_v[pl.ds(i * 16, 16)] = zero16
        return 0
    lax.fori_loop(0, SEG // 16, z16, 0)
    tmps = (tmp_v, tmp2_v)
    csems = (csem0, csem1)
    pltpu.async_copy(pres_sh.at[0, 0, pl.ds(s * SEG, SEG)], tmp_v, csem0)
    for t in range(NS):
        pltpu.make_async_copy(pres_sh.at[t, 0, pl.ds(s * SEG, SEG)],
                              tmps[t % 2], csems[t % 2]).wait()
        if t + 1 < NS:
            pltpu.async_copy(pres_sh.at[t + 1, 0, pl.ds(s * SEG, SEG)],
                             tmps[(t + 1) % 2], csems[(t + 1) % 2])
        tref = tmps[t % 2]

        def add16(k, _, tref=tref):
            acc_v[pl.ds(k * 16, 16)] = (acc_v[pl.ds(k * 16, 16)]
                                        + tref[pl.ds(k * 16, 16)])
            return 0
        lax.fori_loop(0, SEG // 16, add16, 0)
    pltpu.sync_copy(acc_v, pres_sh.at[NS, 0, pl.ds(s * SEG, SEG)])
    plsc.subcore_barrier()
    pltpu.sync_copy(pres_sh.at[NS, 0], pres_v)

    def rank_body(i, carry):
        v = pres_v[pl.ds(i * 16, 16)]
        ind = (v > 0).astype(jnp.int32)
        incl = plsc.cumsum(ind)
        rank_v[pl.ds(i * 16, 16)] = carry + incl - ind
        return carry + jnp.sum(ind)
    lax.fori_loop(0, B // 16, rank_body, jnp.int32(0))

    def edge_body(i, cnt):
        for u in range(2):
            sv = src_v[pl.ds(i * 32 + u * 16, 16)]
            dv = dst_v[pl.ds(i * 32 + u * 16, 16)]
            m = (sv < B) & (dv < B)
            svc = jnp.where(m, sv, 0)
            dvc = jnp.where(m, dv, 0)
            ns = plsc.load_gather(rank_v, [svc], mask=m)
            nd = plsc.load_gather(rank_v, [dvc], mask=m)
            plsc.addupdate_scatter(deg_v, [nd], one16, mask=m)
            mi = m.astype(jnp.int32)
            pos = cnt + plsc.cumsum(mi) - mi
            pos2 = ((pos >> 7) << 8) | (pos & 127)
            plsc.store_scatter(idx_v, [pos2], ns, mask=m)
            plsc.store_scatter(idx_v, [pos2 + KB], nd, mask=m)
            cnt = cnt + jnp.sum(mi)
        return cnt
    cnt = lax.fori_loop(0, CE // 32, edge_body, jnp.int32(0))

    dum = jnp.full((16,), B, jnp.int32)
    for j in range(8):
        idx = cnt + j * 16 + iota16
        idx2 = ((idx >> 7) << 8) | (idx & 127)
        plsc.store_scatter(idx_v, [idx2], dum)
        plsc.store_scatter(idx_v, [idx2 + KB], dum)
    nb = (cnt + KB - 1) // KB
    misc_v[...] = jnp.full((16,), nb, jnp.int32)

    pltpu.sync_copy(idx_v, idx_out.at[w, 0])
    pltpu.sync_copy(misc_v, cnt_out.at[w, 0])
    pltpu.sync_copy(deg_v, deg_out.at[w, 0])


@functools.partial(
    pl.kernel,
    out_type=jax.ShapeDtypeStruct((NC, B, 1, D), jnp.float32),
    mesh=_mesh,
    compiler_params=pltpu.CompilerParams(needs_layout_passes=False),
    scratch_types=(
        pltpu.VMEM((16,), jnp.int32),
        pltpu.VMEM((2 * KB,), jnp.int32),
        pltpu.VMEM((2 * KB,), jnp.int32),
        pltpu.VMEM((2 * KB,), jnp.int32),
        pltpu.VMEM((KB, 1, D), jnp.float32),
        pltpu.VMEM((KB, 1, D), jnp.float32),
        pltpu.VMEM((KB, 1, D), jnp.float32),
        pltpu.VMEM_SHARED((TR, 1, D), jnp.float32),
        pltpu.VMEM_SHARED((TR, 1, D), jnp.float32),
        pltpu.SemaphoreType.DMA,
        pltpu.SemaphoreType.DMA,
        pltpu.SemaphoreType.DMA,
        pltpu.SemaphoreType.DMA,
        pltpu.SemaphoreType.DMA,
        pltpu.SemaphoreType.DMA,
        pltpu.SemaphoreType.DMA,
        pltpu.SemaphoreType.DMA,
        pltpu.SemaphoreType.DMA,
    ),
)
def _conv(ytab, idx_in, cnt_in, zeros, out,
          cnt_v, x0, x1, x2, r0, r1, r2, ytab_sh, acc_sh,
          i0, i1, i2, g0, g1, g2, s0, s1, s2):
    NBUF = 3
    idxb = (x0, x1, x2)
    rows = (r0, r1, r2)
    isems = (i0, i1, i2)
    gsems = (g0, g1, g2)
    ssems = (s0, s1, s2)
    c = lax.axis_index("c")
    s = lax.axis_index("s")
    w = c * NS + s
    TSTR = TR // NS
    pltpu.sync_copy(zeros.at[pl.ds(s * STRIPE, STRIPE)],
                    acc_sh.at[pl.ds(s * STRIPE, STRIPE)])
    pltpu.sync_copy(ytab.at[pl.ds(c * TR + s * TSTR, TSTR)],
                    ytab_sh.at[pl.ds(s * TSTR, TSTR)])
    pltpu.sync_copy(cnt_in.at[w, 0], cnt_v)
    nb = jnp.max(cnt_v[pl.ds(0, 16)])
    plsc.subcore_barrier()

    for b in range(NBUF):
        @pl.when(b < nb)
        def _(b=b):
            pltpu.async_copy(idx_in.at[w, 0, pl.ds(b * 2 * KB, 2 * KB)],
                             idxb[b], isems[b])

    def outer(i, _):
        j0 = i * NBUF
        for b in range(NBUF):
            j = j0 + b

            @pl.when(j < nb)
            def _(b=b, j=j):
                pltpu.make_async_copy(idx_in.at[w, 0, pl.ds(0, 2 * KB)],
                                      idxb[b], isems[b]).wait()
                pltpu.async_copy(ytab_sh.at[idxb[b].at[pl.ds(0, KB)]],
                                 rows[b], gsems[b]).wait()
                pltpu.async_copy(rows[b],
                                 acc_sh.at[idxb[b].at[pl.ds(KB, KB)]],
                                 ssems[b], add=True).wait()

                @pl.when(j + NBUF < nb)
                def _():
                    pltpu.async_copy(
                        idx_in.at[w, 0, pl.ds((j + NBUF) * 2 * KB, 2 * KB)],
                        idxb[b], isems[b])
        return 0
    lax.fori_loop(0, (nb + NBUF - 1) // NBUF, outer, 0)
    plsc.subcore_barrier()
    pltpu.sync_copy(acc_sh.at[pl.ds(s * STRIPE, STRIPE)],
                    out.at[c, pl.ds(s * STRIPE, STRIPE)])


def _tc_a_body(x_ref, w1_ref, degp_ref, ytab_ref, xw_ref, dinv_ref):
    degs = jnp.sum(degp_ref[...].reshape(R, NS, B), axis=1)
    deg = degs.astype(jnp.float32) + 1.0
    dinv = lax.rsqrt(deg)
    dinv_ref[...] = dinv
    zpad = jnp.zeros((TR - B, D), jnp.float32)
    for r in range(R):
        xw = jnp.dot(x_ref[r], w1_ref[...], preferred_element_type=jnp.float32)
        xw_ref[r] = xw
        ytab_ref[pl.ds(r * TR, B), :] = xw * dinv[r][:, None]
        ytab_ref[pl.ds(r * TR + B, TR - B), :] = zpad


def _tc_b_body(acc_ref, xw1_ref, dinv_ref, b1_ref, g_ref, be_ref, w2_ref,
               ytab_ref, xw2_ref):
    zpad = jnp.zeros((TR - B, D), jnp.float32)
    for r in range(R):
        dinv = dinv_ref[r]
        f1 = (acc_ref[r] * dinv[:, None]
              + xw1_ref[r] * (dinv * dinv)[:, None] + b1_ref[...][None, :])
        mu = jnp.mean(f1, axis=0)
        cen = f1 - mu[None, :]
        var = jnp.mean(cen * cen, axis=0)
        f1n = cen * lax.rsqrt(var + 1e-5)[None, :] * g_ref[...][None, :] \
            + be_ref[...][None, :]
        xw2 = jnp.dot(f1n, w2_ref[...], preferred_element_type=jnp.float32)
        xw2_ref[r] = xw2
        ytab_ref[pl.ds(r * TR, B), :] = xw2 * dinv[:, None]
        ytab_ref[pl.ds(r * TR + B, TR - B), :] = zpad


def _tc_c_body(acc_ref, xw2_ref, dinv_ref, b2_ref, out_ref):
    for r in range(R):
        dinv = dinv_ref[r]
        out_ref[r] = (acc_ref[r] * dinv[:, None]
                      + xw2_ref[r] * (dinv * dinv)[:, None]
                      + b2_ref[...][None, :])


_tc_a = pl.pallas_call(
    _tc_a_body,
    out_shape=(
        jax.ShapeDtypeStruct((R * TR, D), jnp.float32),
        jax.ShapeDtypeStruct((R, B, D), jnp.float32),
        jax.ShapeDtypeStruct((R, B), jnp.float32),
    ),
)

_tc_b = pl.pallas_call(
    _tc_b_body,
    out_shape=(
        jax.ShapeDtypeStruct((R * TR, D), jnp.float32),
        jax.ShapeDtypeStruct((R, B, D), jnp.float32),
    ),
)

_tc_c = pl.pallas_call(
    _tc_c_body,
    out_shape=jax.ShapeDtypeStruct((R, B, D), jnp.float32),
)


def kernel(features_list, multi_r_data, batch_nodes, device,
           W1, b1, gamma, beta, W2, b2):
    del batch_nodes, device
    x2 = features_list[:, :B, :]
    edges = multi_r_data.reshape(2 * R, NS, 1, CE)
    idx, cnt, degp = _preprocess(edges)
    ytab1, xw1, dinv = _tc_a(x2, W1, degp)
    zeros = jnp.zeros((B, 1, D), jnp.float32)
    acc1 = _conv(ytab1.reshape(R * TR, 1, D), idx, cnt, zeros)
    ytab2, xw2 = _tc_b(acc1.reshape(R, B, D), xw1, dinv, b1, gamma, beta, W2)
    acc2 = _conv(ytab2.reshape(R * TR, 1, D), idx, cnt, zeros)
    f2 = _tc_c(acc2.reshape(R, B, D), xw2, dinv, b2)
    return f2.reshape(B, R * D)


# speedup vs baseline: 1.2834x; 1.0098x over previous
import functools

import jax
import jax.numpy as jnp
from jax import lax
from jax.experimental import pallas as pl
from jax.experimental.pallas import tpu as pltpu
from jax.experimental.pallas import tpu_sc as plsc

B = 4096
D = 128
E = 320000
R = 2
NC = 2
NS = 16
L = 16
CE = E // NS
KB = 128
CEB = CE + 2 * KB
TR = 4112
STRIPE = B // NS
NW = NC * NS

_mesh = plsc.VectorSubcoreMesh(
    core_axis_name="c", subcore_axis_name="s", num_cores=NC, num_subcores=NS)


@functools.partial(
    pl.kernel,
    out_type=(
        jax.ShapeDtypeStruct((NW, 1, 2 * CEB), jnp.int32),
        jax.ShapeDtypeStruct((NW, 1, 16), jnp.int32),
        jax.ShapeDtypeStruct((NW, 1, B), jnp.int32),
    ),
    mesh=_mesh,
    compiler_params=pltpu.CompilerParams(needs_layout_passes=False),
    scratch_types=(
        pltpu.VMEM((CE,), jnp.int32),
        pltpu.VMEM((CE,), jnp.int32),
        pltpu.VMEM((2 * CEB,), jnp.int32),
        pltpu.VMEM((B,), jnp.int32),
        pltpu.VMEM((B,), jnp.int32),
        pltpu.VMEM((B,), jnp.int32),
        pltpu.VMEM((B // NS,), jnp.int32),
        pltpu.VMEM((B // NS,), jnp.int32),
        pltpu.VMEM((B // NS,), jnp.int32),
        pltpu.VMEM((16,), jnp.int32),
        pltpu.VMEM_SHARED((NS + 1, 1, B), jnp.int32),
        pltpu.SemaphoreType.DMA,
        pltpu.SemaphoreType.DMA,
    ),
)
def _preprocess(edges, idx_out, cnt_out, deg_out,
                src_v, dst_v, idx_v, pres_v, deg_v, rank_v, tmp_v,
                tmp2_v, acc_v, misc_v, pres_sh, csem0, csem1):
    c = lax.axis_index("c")
    s = lax.axis_index("s")
    w = c * NS + s
    iota16 = lax.iota(jnp.int32, 16)
    one16 = jnp.ones((L,), jnp.int32)
    zero16 = jnp.zeros((L,), jnp.int32)

    e0 = pltpu.async_copy(edges.at[2 * c, s, 0], src_v, csem0)
    e1 = pltpu.async_copy(edges.at[2 * c + 1, s, 0], dst_v, csem1)

    def zero_body(i, _):
        pres_v[pl.ds(i * 16, 16)] = zero16
        deg_v[pl.ds(i * 16, 16)] = zero16
        return 0
    lax.fori_loop(0, B // 16, zero_body, 0)
    e0.wait()
    e1.wait()

    def pres_body(i, _):
        for u in range(2):
            sv = src_v[pl.ds(i * 32 + u * 16, 16)]
            dv = dst_v[pl.ds(i * 32 + u * 16, 16)]
            m = (sv < B) & (dv < B)
            svc = jnp.where(m, sv, 0)
            dvc = jnp.where(m, dv, 0)
            plsc.store_scatter(pres_v, [svc], one16, mask=m)
            plsc.store_scatter(pres_v, [dvc], one16, mask=m)
        return 0
    lax.fori_loop(0, CE // 32, pres_body, 0)

    SEG = B // NS
    pltpu.sync_copy(pres_v, pres_sh.at[s, 0])
    plsc.subcore_barrier()

    def z16(i, _):
        acc_v[pl.ds(i * 16, 16)] = zero16
        return 0
    lax.fori_loop(0, SEG // 16, z16, 0)
    tmps = (tmp_v, tmp2_v)
    csems = (csem0, csem1)
    pltpu.async_copy(pres_sh.at[0, 0, pl.ds(s * SEG, SEG)], tmp_v, csem0)
    for t in range(NS):
        pltpu.make_async_copy(pres_sh.at[t, 0, pl.ds(s * SEG, SEG)],
                              tmps[t % 2], csems[t % 2]).wait()
        if t + 1 < NS:
            pltpu.async_copy(pres_sh.at[t + 1, 0, pl.ds(s * SEG, SEG)],
                             tmps[(t + 1) % 2], csems[(t + 1) % 2])
        tref = tmps[t % 2]

        def add16(k, _, tref=tref):
            acc_v[pl.ds(k * 16, 16)] = (acc_v[pl.ds(k * 16, 16)]
                                        + tref[pl.ds(k * 16, 16)])
            return 0
        lax.fori_loop(0, SEG // 16, add16, 0)
    pltpu.sync_copy(acc_v, pres_sh.at[NS, 0, pl.ds(s * SEG, SEG)])
    plsc.subcore_barrier()
    pltpu.sync_copy(pres_sh.at[NS, 0], pres_v)

    def rank_body(i, carry):
        v = pres_v[pl.ds(i * 16, 16)]
        ind = (v > 0).astype(jnp.int32)
        incl = plsc.cumsum(ind)
        rank_v[pl.ds(i * 16, 16)] = carry + incl - ind
        return carry + jnp.sum(ind)
    lax.fori_loop(0, B // 16, rank_body, jnp.int32(0))

    def edge_body(i, cnt):
        for u in range(2):
            sv = src_v[pl.ds(i * 32 + u * 16, 16)]
            dv = dst_v[pl.ds(i * 32 + u * 16, 16)]
            m = (sv < B) & (dv < B)
            svc = jnp.where(m, sv, 0)
            dvc = jnp.where(m, dv, 0)
            ns = plsc.load_gather(rank_v, [svc], mask=m)
            nd = plsc.load_gather(rank_v, [dvc], mask=m)
            plsc.addupdate_scatter(deg_v, [nd], one16, mask=m)
            mi = m.astype(jnp.int32)
            pos = cnt + plsc.cumsum(mi) - mi
            pos2 = ((pos >> 7) << 8) | (pos & 127)
            plsc.store_scatter(idx_v, [pos2], ns, mask=m)
            plsc.store_scatter(idx_v, [pos2 + KB], nd, mask=m)
            cnt = cnt + jnp.sum(mi)
        return cnt
    cnt = lax.fori_loop(0, CE // 32, edge_body, jnp.int32(0))

    dum = jnp.full((16,), B, jnp.int32)
    for j in range(8):
        idx = cnt + j * 16 + iota16
        idx2 = ((idx >> 7) << 8) | (idx & 127)
        plsc.store_scatter(idx_v, [idx2], dum)
        plsc.store_scatter(idx_v, [idx2 + KB], dum)
    nb = (cnt + KB - 1) // KB
    misc_v[...] = jnp.full((16,), nb, jnp.int32)

    pltpu.sync_copy(idx_v, idx_out.at[w, 0])
    pltpu.sync_copy(misc_v, cnt_out.at[w, 0])
    pltpu.sync_copy(deg_v, deg_out.at[w, 0])


@functools.partial(
    pl.kernel,
    out_type=jax.ShapeDtypeStruct((NC, B, 1, D), jnp.float32),
    mesh=_mesh,
    compiler_params=pltpu.CompilerParams(needs_layout_passes=False),
    scratch_types=(
        pltpu.VMEM((16,), jnp.int32),
        pltpu.VMEM((2 * KB,), jnp.int32),
        pltpu.VMEM((2 * KB,), jnp.int32),
        pltpu.VMEM((2 * KB,), jnp.int32),
        pltpu.VMEM((KB, 1, D), jnp.float32),
        pltpu.VMEM((KB, 1, D), jnp.float32),
        pltpu.VMEM((KB, 1, D), jnp.float32),
        pltpu.VMEM_SHARED((TR, 1, D), jnp.float32),
        pltpu.VMEM_SHARED((TR, 1, D), jnp.float32),
        pltpu.SemaphoreType.DMA,
        pltpu.SemaphoreType.DMA,
        pltpu.SemaphoreType.DMA,
        pltpu.SemaphoreType.DMA,
        pltpu.SemaphoreType.DMA,
        pltpu.SemaphoreType.DMA,
        pltpu.SemaphoreType.DMA,
        pltpu.SemaphoreType.DMA,
        pltpu.SemaphoreType.DMA,
    ),
)
def _conv(ytab, idx_in, cnt_in, zeros, out,
          cnt_v, x0, x1, x2, r0, r1, r2, ytab_sh, acc_sh,
          i0, i1, i2, g0, g1, g2, s0, s1, s2):
    NBUF = 3
    idxb = (x0, x1, x2)
    rows = (r0, r1, r2)
    isems = (i0, i1, i2)
    gsems = (g0, g1, g2)
    ssems = (s0, s1, s2)
    c = lax.axis_index("c")
    s = lax.axis_index("s")
    w = c * NS + s
    TSTR = TR // NS
    z = pltpu.async_copy(zeros.at[pl.ds(s * STRIPE, STRIPE)],
                         acc_sh.at[pl.ds(s * STRIPE, STRIPE)], g0)
    t = pltpu.async_copy(ytab.at[pl.ds(c * TR + s * TSTR, TSTR)],
                         ytab_sh.at[pl.ds(s * TSTR, TSTR)], g1)
    pltpu.sync_copy(cnt_in.at[w, 0], cnt_v)
    nb = jnp.max(cnt_v[pl.ds(0, 16)])
    z.wait()
    t.wait()
    plsc.subcore_barrier()

    for b in range(NBUF):
        @pl.when(b < nb)
        def _(b=b):
            pltpu.async_copy(idx_in.at[w, 0, pl.ds(b * 2 * KB, 2 * KB)],
                             idxb[b], isems[b])

    def outer(i, _):
        j0 = i * NBUF
        for b in range(NBUF):
            j = j0 + b

            @pl.when(j < nb)
            def _(b=b, j=j):
                pltpu.make_async_copy(idx_in.at[w, 0, pl.ds(0, 2 * KB)],
                                      idxb[b], isems[b]).wait()
                pltpu.async_copy(ytab_sh.at[idxb[b].at[pl.ds(0, KB)]],
                                 rows[b], gsems[b]).wait()
                pltpu.async_copy(rows[b],
                                 acc_sh.at[idxb[b].at[pl.ds(KB, KB)]],
                                 ssems[b], add=True).wait()

                @pl.when(j + NBUF < nb)
                def _():
                    pltpu.async_copy(
                        idx_in.at[w, 0, pl.ds((j + NBUF) * 2 * KB, 2 * KB)],
                        idxb[b], isems[b])
        return 0
    lax.fori_loop(0, (nb + NBUF - 1) // NBUF, outer, 0)
    plsc.subcore_barrier()
    pltpu.sync_copy(acc_sh.at[pl.ds(s * STRIPE, STRIPE)],
                    out.at[c, pl.ds(s * STRIPE, STRIPE)])


def _tc_a_body(x_ref, w1_ref, degp_ref, ytab_ref, xw_ref, dinv_ref):
    degs = jnp.sum(degp_ref[...].reshape(R, NS, B), axis=1)
    deg = degs.astype(jnp.float32) + 1.0
    dinv = lax.rsqrt(deg)
    dinv_ref[...] = dinv
    zpad = jnp.zeros((TR - B, D), jnp.float32)
    for r in range(R):
        xw = jnp.dot(x_ref[r], w1_ref[...], preferred_element_type=jnp.float32)
        xw_ref[r] = xw
        ytab_ref[pl.ds(r * TR, B), :] = xw * dinv[r][:, None]
        ytab_ref[pl.ds(r * TR + B, TR - B), :] = zpad


def _tc_b_body(acc_ref, xw1_ref, dinv_ref, b1_ref, g_ref, be_ref, w2_ref,
               ytab_ref, xw2_ref):
    zpad = jnp.zeros((TR - B, D), jnp.float32)
    for r in range(R):
        dinv = dinv_ref[r]
        f1 = (acc_ref[r] * dinv[:, None]
              + xw1_ref[r] * (dinv * dinv)[:, None] + b1_ref[...][None, :])
        mu = jnp.mean(f1, axis=0)
        cen = f1 - mu[None, :]
        var = jnp.mean(cen * cen, axis=0)
        f1n = cen * lax.rsqrt(var + 1e-5)[None, :] * g_ref[...][None, :] \
            + be_ref[...][None, :]
        xw2 = jnp.dot(f1n, w2_ref[...], preferred_element_type=jnp.float32)
        xw2_ref[r] = xw2
        ytab_ref[pl.ds(r * TR, B), :] = xw2 * dinv[:, None]
        ytab_ref[pl.ds(r * TR + B, TR - B), :] = zpad


def _tc_c_body(acc_ref, xw2_ref, dinv_ref, b2_ref, out_ref):
    for r in range(R):
        dinv = dinv_ref[r]
        out_ref[r] = (acc_ref[r] * dinv[:, None]
                      + xw2_ref[r] * (dinv * dinv)[:, None]
                      + b2_ref[...][None, :])


_tc_a = pl.pallas_call(
    _tc_a_body,
    out_shape=(
        jax.ShapeDtypeStruct((R * TR, D), jnp.float32),
        jax.ShapeDtypeStruct((R, B, D), jnp.float32),
        jax.ShapeDtypeStruct((R, B), jnp.float32),
    ),
)

_tc_b = pl.pallas_call(
    _tc_b_body,
    out_shape=(
        jax.ShapeDtypeStruct((R * TR, D), jnp.float32),
        jax.ShapeDtypeStruct((R, B, D), jnp.float32),
    ),
)

_tc_c = pl.pallas_call(
    _tc_c_body,
    out_shape=jax.ShapeDtypeStruct((R, B, D), jnp.float32),
)


def kernel(features_list, multi_r_data, batch_nodes, device,
           W1, b1, gamma, beta, W2, b2):
    del batch_nodes, device
    x2 = features_list[:, :B, :]
    edges = multi_r_data.reshape(2 * R, NS, 1, CE)
    idx, cnt, degp = _preprocess(edges)
    ytab1, xw1, dinv = _tc_a(x2, W1, degp)
    zeros = jnp.zeros((B, 1, D), jnp.float32)
    acc1 = _conv(ytab1.reshape(R * TR, 1, D), idx, cnt, zeros)
    ytab2, xw2 = _tc_b(acc1.reshape(R, B, D), xw1, dinv, b1, gamma, beta, W2)
    acc2 = _conv(ytab2.reshape(R * TR, 1, D), idx, cnt, zeros)
    f2 = _tc_c(acc2.reshape(R, B, D), xw2, dinv, b2)
    return f2.reshape(B, R * D)
